# baseline (speedup 1.0000x reference)
import jax
import jax.numpy as jnp
from jax.experimental import pallas as pl
from jax.experimental.pallas import tpu as pltpu

LB_SMOOTH_ = 0.1
IGNORE_INDEX_ = 255
H_BLOCK = 128
SUB = 32


def _ce_kernel(x_ref, lab_ref, loss_ref, cnt_ref):
    h = pl.program_id(1)
    num_classes = x_ref.shape[1]
    w = x_ref.shape[3]

    lb_pos = 1.0 - LB_SMOOTH_
    lb_neg = LB_SMOOTH_ / num_classes
    k_const = lb_pos + (num_classes - 1) * lb_neg

    def body(r, accs):
        loss_acc, cnt_acc = accs
        row = r * SUB
        lab = lab_ref[0, pl.ds(row, SUB), :]
        ignore = lab == IGNORE_INDEX_

        s0 = jnp.zeros((SUB, w), jnp.float32)
        s1 = jnp.zeros((SUB, w), jnp.float32)
        w0 = jnp.zeros((SUB, w), jnp.float32)
        w1 = jnp.zeros((SUB, w), jnp.float32)
        for c in range(num_classes):
            xc = x_ref[0, c, pl.ds(row, SUB), :]
            wc = jnp.where(lab == c, lb_pos, lb_neg)
            if c % 2 == 0:
                s0 = s0 + jnp.exp(xc)
                w0 = w0 + wc * xc
            else:
                s1 = s1 + jnp.exp(xc)
                w1 = w1 + wc * xc

        wsum = w0 + w1
        lse = jnp.log(s0 + s1)
        loss = k_const * lse - wsum
        loss = jnp.where(ignore, 0.0, loss)
        loss_acc = loss_acc + loss
        cnt_acc = cnt_acc + jnp.where(ignore, 0.0, 1.0)
        return loss_acc, cnt_acc

    z = jnp.zeros((SUB, w), jnp.float32)
    loss_acc, cnt_acc = jax.lax.fori_loop(
        0, H_BLOCK // SUB, body, (z, z), unroll=False
    )
    part = jnp.sum(loss_acc).reshape(1, 1, 1)
    cnt = jnp.sum(cnt_acc).reshape(1, 1, 1)

    @pl.when(h == 0)
    def _init():
        loss_ref[...] = part
        cnt_ref[...] = cnt

    @pl.when(h != 0)
    def _acc():
        loss_ref[...] += part
        cnt_ref[...] += cnt


def kernel(logits, label):
    n, c, hh, w = logits.shape
    label = label.astype(jnp.int32)
    grid = (n, hh // H_BLOCK)

    loss_sums, cnts = pl.pallas_call(
        _ce_kernel,
        grid=grid,
        in_specs=[
            pl.BlockSpec((1, c, H_BLOCK, w), lambda i, j: (i, 0, j, 0)),
            pl.BlockSpec((1, H_BLOCK, w), lambda i, j: (i, j, 0)),
        ],
        out_specs=[
            pl.BlockSpec((1, 1, 1), lambda i, j: (i, 0, 0)),
            pl.BlockSpec((1, 1, 1), lambda i, j: (i, 0, 0)),
        ],
        out_shape=[
            jax.ShapeDtypeStruct((n, 1, 1), jnp.float32),
            jax.ShapeDtypeStruct((n, 1, 1), jnp.float32),
        ],
        compiler_params=pltpu.CompilerParams(
            dimension_semantics=("parallel", "arbitrary"),
        ),
    )(logits.astype(jnp.float32), label)

    return jnp.sum(loss_sums) / jnp.sum(cnts)
